# baseline (speedup 1.0000x reference)
import functools

import jax
import jax.numpy as jnp
from jax import lax
from jax.experimental import pallas as pl
from jax.experimental.pallas import tpu as pltpu
from jax.experimental.pallas import tpu_sc as plsc

DIM = 1024
W = 16
NBUF = 4


def kernel(input_ids, embedding_table):
    batch, seq = input_ids.shape
    n = batch * seq
    idx = input_ids.reshape(n).astype(jnp.int32)

    NC, NS = 2, 16
    NW = NC * NS
    b_per_w = n // NW
    n_chunks = b_per_w // W

    mesh = plsc.VectorSubcoreMesh(core_axis_name="c", subcore_axis_name="s")

    @functools.partial(
        pl.kernel,
        out_type=jax.ShapeDtypeStruct((n, DIM), embedding_table.dtype),
        mesh=mesh,
        scratch_types=[
            pltpu.VMEM((b_per_w,), jnp.int32),
            pltpu.VMEM((NBUF, W, DIM), jnp.float32),
            pltpu.SemaphoreType.DMA((NBUF,)),
            pltpu.SemaphoreType.DMA((NBUF,)),
        ],
    )
    def gather_kernel(table_hbm, idx_hbm, out_hbm, idx_v, rows_v, gsem, wsem):
        wid = lax.axis_index("s") * NC + lax.axis_index("c")
        base = wid * b_per_w
        pltpu.sync_copy(idx_hbm.at[pl.ds(base, b_per_w)], idx_v)

        def g_copy(c, b):
            return pltpu.make_async_copy(
                table_hbm.at[idx_v.at[pl.ds(c * W, W)]],
                rows_v.at[b], gsem.at[b])

        def w_copy(c, b):
            return pltpu.make_async_copy(
                rows_v.at[b], out_hbm.at[pl.ds(base + c * W, W)], wsem.at[b])

        g_copy(0, 0).start()

        @pl.loop(0, n_chunks, step=NBUF)
        def _(c0):
            for j in range(NBUF):
                c = c0 + j
                g_copy(c, j).wait()
                w_copy(c, j).start()
                nb = (j + 1) % NBUF

                @pl.when(c + 1 >= NBUF)
                def _():
                    w_copy(c + 1 - NBUF, nb).wait()

                @pl.when(c + 1 < n_chunks)
                def _():
                    g_copy(c + 1, nb).start()

        for c in range(n_chunks - NBUF + 1, n_chunks):
            w_copy(c, c % NBUF).wait()

    out = gather_kernel(embedding_table, idx)
    return out.reshape(batch, seq, DIM)
